# Initial kernel scaffold; baseline (speedup 1.0000x reference)
import jax
import jax.numpy as jnp
from jax.experimental import pallas as pl


def kernel(lookup_ids, embedding_matrix):
    raise NotImplementedError("write your pallas kernel here")


# speedup vs baseline: 2.9422x; 2.9422x over previous
import functools

import jax
import jax.numpy as jnp
from jax import lax
from jax.experimental import pallas as pl
from jax.experimental.pallas import tpu as pltpu
from jax.experimental.pallas import tpu_sc as plsc

PAD_ID = 0
EOS_ID = 1

NUM_CORES = 2
NUM_SUBCORES = 16
NW = NUM_CORES * NUM_SUBCORES

CHUNK = 128


def _make_sc_gather(total_rows: int, dim: int):
    assert total_rows % (NW * CHUNK) == 0
    rows_per_w = total_rows // NW
    n_chunks = rows_per_w // CHUNK

    mesh = plsc.VectorSubcoreMesh(core_axis_name="c", subcore_axis_name="s")

    @functools.partial(
        pl.kernel,
        mesh=mesh,
        out_type=jax.ShapeDtypeStruct((total_rows, dim), jnp.float32),
        scratch_types=[
            pltpu.VMEM((n_chunks, CHUNK), jnp.int32),
            pltpu.VMEM((CHUNK, dim), jnp.float32),
            pltpu.SemaphoreType.DMA,
        ],
    )
    def sc_gather(table_hbm, idx_hbm, out_hbm, idx_v, rows_v, gsem):
        wid = lax.axis_index("s") * NUM_CORES + lax.axis_index("c")
        base = wid * rows_per_w
        pltpu.sync_copy(idx_hbm.at[wid], idx_v)

        def body(j, carry):
            pltpu.async_copy(table_hbm.at[idx_v.at[j]], rows_v, gsem).wait()
            pltpu.sync_copy(rows_v, out_hbm.at[pl.ds(base + j * CHUNK, CHUNK)])
            return carry

        lax.fori_loop(0, n_chunks, body, 0)

    return sc_gather


def _mask_eos_kernel(ids_ref, mask_ref, eos_ref):
    ids = ids_ref[...]
    mask_ref[...] = ids == PAD_ID
    eos_ref[...] = (ids == EOS_ID).astype(jnp.float32)


def kernel(lookup_ids, embedding_matrix):
    b, s = lookup_ids.shape
    v, d = embedding_matrix.shape
    total = b * s

    idx3 = lookup_ids.reshape(NW, total // (NW * CHUNK), CHUNK)
    gather = _make_sc_gather(total, d)
    flat = gather(embedding_matrix, idx3)
    matrices = flat.reshape(b, s, d)

    mask, eos = pl.pallas_call(
        _mask_eos_kernel,
        out_shape=(
            jax.ShapeDtypeStruct((b, s), jnp.bool_),
            jax.ShapeDtypeStruct((b, s), jnp.float32),
        ),
    )(lookup_ids)

    return matrices, mask, eos, lookup_ids


# speedup vs baseline: 3.3133x; 1.1261x over previous
import functools

import jax
import jax.numpy as jnp
from jax import lax
from jax.experimental import pallas as pl
from jax.experimental.pallas import tpu as pltpu
from jax.experimental.pallas import tpu_sc as plsc

PAD_ID = 0
EOS_ID = 1

NUM_CORES = 2
NUM_SUBCORES = 16
NW = NUM_CORES * NUM_SUBCORES

CHUNK = 128
RING = 5
LEAD = 3


def _make_sc_gather(total_rows: int, dim: int):
    assert total_rows % (NW * CHUNK) == 0
    rows_per_w = total_rows // NW
    n_chunks = rows_per_w // CHUNK
    assert n_chunks % RING == 0 and n_chunks >= 2 * RING

    mesh = plsc.VectorSubcoreMesh(core_axis_name="c", subcore_axis_name="s")

    @functools.partial(
        pl.kernel,
        mesh=mesh,
        out_type=jax.ShapeDtypeStruct((total_rows, dim), jnp.float32),
        scratch_types=(
            [pltpu.VMEM((n_chunks, CHUNK), jnp.int32)]
            + [pltpu.VMEM((CHUNK, dim), jnp.float32)] * RING
            + [pltpu.SemaphoreType.DMA] * (2 * RING)
        ),
    )
    def sc_gather(table_hbm, idx_hbm, out_hbm, idx_v, *rest):
        bufs = rest[:RING]
        gsems = rest[RING:2 * RING]
        osems = rest[2 * RING:]

        wid = lax.axis_index("s") * NUM_CORES + lax.axis_index("c")
        base = wid * rows_per_w
        pltpu.sync_copy(idx_hbm.at[wid], idx_v)

        def fire_gather(j, b):
            pltpu.async_copy(table_hbm.at[idx_v.at[j]], bufs[b], gsems[b])

        def wait_gather(b):
            pltpu.make_async_copy(
                table_hbm.at[idx_v.at[0]], bufs[b], gsems[b]).wait()

        def fire_out(j, b):
            pltpu.async_copy(
                bufs[b], out_hbm.at[pl.ds(base + j * CHUNK, CHUNK)], osems[b])

        def wait_out(b):
            pltpu.make_async_copy(
                bufs[b], out_hbm.at[pl.ds(base, CHUNK)], osems[b]).wait()

        for j in range(LEAD):
            fire_gather(j, j % RING)

        def step(t, b, first_use, fire_ok):
            j = RING * t + b
            bf = (b + LEAD) % RING
            if fire_ok:
                if not first_use:
                    wait_out(bf)
                fire_gather(j + LEAD, bf)
            wait_gather(b)
            fire_out(j, b)

        for b in range(RING):
            step(0, b, first_use=(b + LEAD < RING), fire_ok=True)

        def body(t, carry):
            for b in range(RING):
                step(t, b, first_use=False, fire_ok=True)
            return carry

        lax.fori_loop(1, n_chunks // RING - 1, body, 0)

        t_last = n_chunks // RING - 1
        for b in range(RING):
            step(t_last, b, first_use=False, fire_ok=(b + LEAD < RING))

        for b in range(RING):
            wait_out(b)

    return sc_gather


def _mask_eos_kernel(ids_ref, mask_ref, eos_ref):
    ids = ids_ref[...]
    mask_ref[...] = ids == PAD_ID
    eos_ref[...] = (ids == EOS_ID).astype(jnp.float32)


def kernel(lookup_ids, embedding_matrix):
    b, s = lookup_ids.shape
    v, d = embedding_matrix.shape
    total = b * s

    idx3 = lookup_ids.reshape(NW, total // (NW * CHUNK), CHUNK)
    gather = _make_sc_gather(total, d)
    flat = gather(embedding_matrix, idx3)
    matrices = flat.reshape(b, s, d)

    mask, eos = pl.pallas_call(
        _mask_eos_kernel,
        out_shape=(
            jax.ShapeDtypeStruct((b, s), jnp.bool_),
            jax.ShapeDtypeStruct((b, s), jnp.float32),
        ),
    )(lookup_ids)

    return matrices, mask, eos, lookup_ids


# speedup vs baseline: 3.3265x; 1.0040x over previous
import functools

import jax
import jax.numpy as jnp
from jax import lax
from jax.experimental import pallas as pl
from jax.experimental.pallas import tpu as pltpu
from jax.experimental.pallas import tpu_sc as plsc

PAD_ID = 0
EOS_ID = 1

NUM_CORES = 2
NUM_SUBCORES = 16
NW = NUM_CORES * NUM_SUBCORES

CHUNK = 128
RING = 5
LEAD = 3


def _make_sc_gather(total_rows: int, dim: int):
    assert total_rows % (NW * CHUNK) == 0
    rows_per_w = total_rows // NW
    n_chunks = rows_per_w // CHUNK
    assert n_chunks % RING == 0 and n_chunks >= 2 * RING
    max_off = max((w * n_chunks) % 8 for w in range(NW))
    idx_window = n_chunks + max_off
    assert all(
        (w * n_chunks // 8) * 8 + idx_window <= NW * n_chunks
        for w in range(NW))

    mesh = plsc.VectorSubcoreMesh(core_axis_name="c", subcore_axis_name="s")

    @functools.partial(
        pl.kernel,
        mesh=mesh,
        out_type=jax.ShapeDtypeStruct((total_rows, dim), jnp.float32),
        compiler_params=pltpu.CompilerParams(use_tc_tiling_on_sc=True),
        scratch_types=(
            [pltpu.VMEM((idx_window, CHUNK), jnp.int32)]
            + [pltpu.VMEM((CHUNK, dim), jnp.float32)] * RING
            + [pltpu.SemaphoreType.DMA] * (2 * RING)
        ),
    )
    def sc_gather(table_hbm, idx_hbm, out_hbm, idx_v, *rest):
        bufs = rest[:RING]
        gsems = rest[RING:2 * RING]
        osems = rest[2 * RING:]

        wid = lax.axis_index("s") * NUM_CORES + lax.axis_index("c")
        base = wid * rows_per_w
        start = wid * n_chunks
        astart = pl.multiple_of((start // 8) * 8, 8)
        off = start - astart
        pltpu.sync_copy(idx_hbm.at[pl.ds(astart, idx_window)], idx_v)

        def fire_gather(j, b):
            pltpu.async_copy(table_hbm.at[idx_v.at[off + j]], bufs[b], gsems[b])

        def wait_gather(b):
            pltpu.make_async_copy(
                table_hbm.at[idx_v.at[0]], bufs[b], gsems[b]).wait()

        def fire_out(j, b):
            pltpu.async_copy(
                bufs[b], out_hbm.at[pl.ds(base + j * CHUNK, CHUNK)], osems[b])

        def wait_out(b):
            pltpu.make_async_copy(
                bufs[b], out_hbm.at[pl.ds(base, CHUNK)], osems[b]).wait()

        for j in range(LEAD):
            fire_gather(j, j % RING)

        def step(t, b, first_use, fire_ok):
            j = RING * t + b
            bf = (b + LEAD) % RING
            if fire_ok:
                if not first_use:
                    wait_out(bf)
                fire_gather(j + LEAD, bf)
            wait_gather(b)
            fire_out(j, b)

        for b in range(RING):
            step(0, b, first_use=(b + LEAD < RING), fire_ok=True)

        def body(t, carry):
            for b in range(RING):
                step(t, b, first_use=False, fire_ok=True)
            return carry

        lax.fori_loop(1, n_chunks // RING - 1, body, 0)

        t_last = n_chunks // RING - 1
        for b in range(RING):
            step(t_last, b, first_use=False, fire_ok=(b + LEAD < RING))

        for b in range(RING):
            wait_out(b)

    return sc_gather


def _mask_eos_kernel(ids_ref, mask_ref, eos_ref):
    ids = ids_ref[...]
    mask_ref[...] = ids == PAD_ID
    eos_ref[...] = (ids == EOS_ID).astype(jnp.float32)


def kernel(lookup_ids, embedding_matrix):
    b, s = lookup_ids.shape
    v, d = embedding_matrix.shape
    total = b * s

    idx3 = lookup_ids.reshape(total // CHUNK, CHUNK)
    gather = _make_sc_gather(total, d)
    flat = gather(embedding_matrix, idx3)
    matrices = flat.reshape(b, s, d)

    mask, eos = pl.pallas_call(
        _mask_eos_kernel,
        out_shape=(
            jax.ShapeDtypeStruct((b, s), jnp.bool_),
            jax.ShapeDtypeStruct((b, s), jnp.float32),
        ),
    )(lookup_ids)

    return matrices, mask, eos, lookup_ids


# speedup vs baseline: 5.7877x; 1.7399x over previous
import functools

import jax
import jax.numpy as jnp
from jax import lax
from jax.experimental import pallas as pl
from jax.experimental.pallas import tpu as pltpu
from jax.experimental.pallas import tpu_sc as plsc

PAD_ID = 0
EOS_ID = 1

NUM_CORES = 2
NUM_SUBCORES = 16
NW = NUM_CORES * NUM_SUBCORES

WCHUNK = 8


def _make_sc_gather(batch: int, seq: int, dim: int):
    assert batch % (NW * WCHUNK) == 0
    b_per_w = batch // NW
    n_blocks = b_per_w // WCHUNK

    mesh = plsc.VectorSubcoreMesh(core_axis_name="c", subcore_axis_name="s")

    @functools.partial(
        pl.kernel,
        mesh=mesh,
        out_type=jax.ShapeDtypeStruct((batch, seq, dim), jnp.float32),
        compiler_params=pltpu.CompilerParams(use_tc_tiling_on_sc=True),
        scratch_types=(
            [pltpu.VMEM((b_per_w, seq), jnp.int32)]
            + [pltpu.VMEM((WCHUNK, seq, dim), jnp.float32)] * 2
            + [pltpu.SemaphoreType.DMA] * 4
        ),
    )
    def sc_gather(table_hbm, ids_hbm, out_hbm, idx_v, buf0, buf1,
                  gsem0, gsem1, osem0, osem1):
        bufs = (buf0, buf1)
        gsems = (gsem0, gsem1)
        osems = (osem0, osem1)

        wid = lax.axis_index("s") * NUM_CORES + lax.axis_index("c")
        b0 = wid * b_per_w
        pltpu.sync_copy(ids_hbm.at[pl.ds(b0, b_per_w)], idx_v)

        def fire_gathers(t, r):
            for k in range(WCHUNK):
                pltpu.async_copy(
                    table_hbm.at[idx_v.at[t * WCHUNK + k]],
                    bufs[r].at[k], gsems[r])

        def drain_gathers(r):
            pltpu.make_async_copy(
                out_hbm.at[pl.ds(0, WCHUNK)], bufs[r], gsems[r]).wait()

        def fire_write(t, r):
            pltpu.async_copy(
                bufs[r], out_hbm.at[pl.ds(b0 + t * WCHUNK, WCHUNK)], osems[r])

        def drain_write(r):
            pltpu.make_async_copy(
                bufs[r], out_hbm.at[pl.ds(0, WCHUNK)], osems[r]).wait()

        fire_gathers(0, 0)

        def half_step(t, r):
            nr = 1 - r
            drain_write(nr)
            fire_gathers(t + 1, nr)
            drain_gathers(r)
            fire_write(t, r)

        fire_gathers(1, 1)
        drain_gathers(0)
        fire_write(0, 0)

        def body(m, carry):
            half_step(2 * m + 1, 1)
            half_step(2 * m + 2, 0)
            return carry

        assert n_blocks % 2 == 0
        lax.fori_loop(0, (n_blocks - 2) // 2, body, 0)

        drain_gathers(1)
        fire_write(n_blocks - 1, 1)
        drain_write(0)
        drain_write(1)

    return sc_gather


def _mask_eos_kernel(ids_ref, mask_ref, eos_ref):
    ids = ids_ref[...]
    mask_ref[...] = ids == PAD_ID
    eos_ref[...] = (ids == EOS_ID).astype(jnp.float32)


def kernel(lookup_ids, embedding_matrix):
    b, s = lookup_ids.shape
    v, d = embedding_matrix.shape

    gather = _make_sc_gather(b, s, d)
    matrices = gather(embedding_matrix, lookup_ids)

    mask, eos = pl.pallas_call(
        _mask_eos_kernel,
        out_shape=(
            jax.ShapeDtypeStruct((b, s), jnp.bool_),
            jax.ShapeDtypeStruct((b, s), jnp.float32),
        ),
    )(lookup_ids)

    return matrices, mask, eos, lookup_ids


# speedup vs baseline: 10.1682x; 1.7569x over previous
import functools

import jax
import jax.numpy as jnp
from jax import lax
from jax.experimental import pallas as pl
from jax.experimental.pallas import tpu as pltpu
from jax.experimental.pallas import tpu_sc as plsc

PAD_ID = 0
EOS_ID = 1

NUM_CORES = 2
NUM_SUBCORES = 16
NW = NUM_CORES * NUM_SUBCORES

CHUNK = 128
RING = 5
LEAD = 3


def _make_sc_gather(total_rows: int, dim: int):
    assert total_rows % (NW * CHUNK) == 0
    rows_per_w = total_rows // NW
    n_chunks = rows_per_w // CHUNK
    assert n_chunks % RING == 0 and n_chunks >= 2 * RING
    max_off = max((w * n_chunks) % 8 for w in range(NW))
    idx_window = n_chunks + max_off
    assert all(
        (w * n_chunks // 8) * 8 + idx_window <= NW * n_chunks
        for w in range(NW))

    mesh = plsc.VectorSubcoreMesh(core_axis_name="c", subcore_axis_name="s")

    @functools.partial(
        pl.kernel,
        mesh=mesh,
        out_type=jax.ShapeDtypeStruct((total_rows, dim), jnp.float32),
        compiler_params=pltpu.CompilerParams(use_tc_tiling_on_sc=True),
        scratch_types=(
            [pltpu.VMEM((idx_window, CHUNK), jnp.int32)]
            + [pltpu.VMEM((CHUNK, dim), jnp.float32)] * RING
            + [pltpu.SemaphoreType.DMA] * (2 * RING)
        ),
    )
    def sc_gather(table_hbm, idx_hbm, out_hbm, idx_v, *rest):
        bufs = rest[:RING]
        gsems = rest[RING:2 * RING]
        osems = rest[2 * RING:]

        wid = lax.axis_index("s") * NUM_CORES + lax.axis_index("c")
        base = wid * rows_per_w
        start = wid * n_chunks
        astart = pl.multiple_of((start // 8) * 8, 8)
        off = start - astart
        pltpu.sync_copy(idx_hbm.at[pl.ds(astart, idx_window)], idx_v)

        def fire_gather(j, b):
            pltpu.async_copy(table_hbm.at[idx_v.at[off + j]], bufs[b], gsems[b])

        def wait_gather(b):
            pltpu.make_async_copy(
                table_hbm.at[idx_v.at[0]], bufs[b], gsems[b]).wait()

        def fire_out(j, b):
            pltpu.async_copy(
                bufs[b], out_hbm.at[pl.ds(base + j * CHUNK, CHUNK)], osems[b])

        def wait_out(b):
            pltpu.make_async_copy(
                bufs[b], out_hbm.at[pl.ds(base, CHUNK)], osems[b]).wait()

        for j in range(LEAD):
            fire_gather(j, j % RING)

        def step(t, b, first_use, fire_ok):
            j = RING * t + b
            bf = (b + LEAD) % RING
            if fire_ok:
                if not first_use:
                    wait_out(bf)
                fire_gather(j + LEAD, bf)
            wait_gather(b)
            fire_out(j, b)

        for b in range(RING):
            step(0, b, first_use=(b + LEAD < RING), fire_ok=True)

        def body(t, carry):
            for b in range(RING):
                step(t, b, first_use=False, fire_ok=True)
            return carry

        lax.fori_loop(1, n_chunks // RING - 1, body, 0)

        t_last = n_chunks // RING - 1
        for b in range(RING):
            step(t_last, b, first_use=False, fire_ok=(b + LEAD < RING))

        for b in range(RING):
            wait_out(b)

    return sc_gather


def _mask_eos_kernel(ids_ref, mask_ref, eos_ref):
    ids = ids_ref[...]
    mask_ref[...] = ids == PAD_ID
    eos_ref[...] = (ids == EOS_ID).astype(jnp.float32)


def kernel(lookup_ids, embedding_matrix):
    b, s = lookup_ids.shape
    v, d = embedding_matrix.shape
    total = b * s

    ids_t = lookup_ids.T
    idx2 = ids_t.reshape(total // CHUNK, CHUNK)
    gather = _make_sc_gather(total, d)
    flat = gather(embedding_matrix, idx2)
    matrices = flat.reshape(s, b, d).transpose(1, 0, 2)

    mask_t, eos_t = pl.pallas_call(
        _mask_eos_kernel,
        out_shape=(
            jax.ShapeDtypeStruct((s, b), jnp.bool_),
            jax.ShapeDtypeStruct((s, b), jnp.float32),
        ),
    )(ids_t)

    return matrices, mask_t.T, eos_t.T, lookup_ids


# speedup vs baseline: 10.1957x; 1.0027x over previous
import functools

import jax
import jax.numpy as jnp
from jax import lax
from jax.experimental import pallas as pl
from jax.experimental.pallas import tpu as pltpu
from jax.experimental.pallas import tpu_sc as plsc

PAD_ID = 0
EOS_ID = 1

NUM_CORES = 2
NUM_SUBCORES = 16
NW = NUM_CORES * NUM_SUBCORES

CHUNK = 128
RING = 5
LEAD = 2


def _make_sc_gather(total_rows: int, dim: int):
    assert total_rows % (NW * CHUNK) == 0
    rows_per_w = total_rows // NW
    n_chunks = rows_per_w // CHUNK
    assert n_chunks % RING == 0 and n_chunks >= 2 * RING
    max_off = max((w * n_chunks) % 8 for w in range(NW))
    idx_window = n_chunks + max_off
    assert all(
        (w * n_chunks // 8) * 8 + idx_window <= NW * n_chunks
        for w in range(NW))

    mesh = plsc.VectorSubcoreMesh(core_axis_name="c", subcore_axis_name="s")

    @functools.partial(
        pl.kernel,
        mesh=mesh,
        out_type=jax.ShapeDtypeStruct((total_rows, dim), jnp.float32),
        compiler_params=pltpu.CompilerParams(use_tc_tiling_on_sc=True),
        scratch_types=(
            [pltpu.VMEM((idx_window, CHUNK), jnp.int32)]
            + [pltpu.VMEM((CHUNK, dim), jnp.float32)] * RING
            + [pltpu.SemaphoreType.DMA] * (2 * RING)
        ),
    )
    def sc_gather(table_hbm, idx_hbm, out_hbm, idx_v, *rest):
        bufs = rest[:RING]
        gsems = rest[RING:2 * RING]
        osems = rest[2 * RING:]

        wid = lax.axis_index("s") * NUM_CORES + lax.axis_index("c")
        base = wid * rows_per_w
        start = wid * n_chunks
        astart = pl.multiple_of((start // 8) * 8, 8)
        off = start - astart
        pltpu.sync_copy(idx_hbm.at[pl.ds(astart, idx_window)], idx_v)

        def fire_gather(j, b):
            pltpu.async_copy(table_hbm.at[idx_v.at[off + j]], bufs[b], gsems[b])

        def wait_gather(b):
            pltpu.make_async_copy(
                table_hbm.at[idx_v.at[0]], bufs[b], gsems[b]).wait()

        def fire_out(j, b):
            pltpu.async_copy(
                bufs[b], out_hbm.at[pl.ds(base + j * CHUNK, CHUNK)], osems[b])

        def wait_out(b):
            pltpu.make_async_copy(
                bufs[b], out_hbm.at[pl.ds(base, CHUNK)], osems[b]).wait()

        for j in range(LEAD):
            fire_gather(j, j % RING)

        def step(t, b, first_use, fire_ok):
            j = RING * t + b
            bf = (b + LEAD) % RING
            if fire_ok:
                if not first_use:
                    wait_out(bf)
                fire_gather(j + LEAD, bf)
            wait_gather(b)
            fire_out(j, b)

        for b in range(RING):
            step(0, b, first_use=(b + LEAD < RING), fire_ok=True)

        def body(t, carry):
            for b in range(RING):
                step(t, b, first_use=False, fire_ok=True)
            return carry

        lax.fori_loop(1, n_chunks // RING - 1, body, 0)

        t_last = n_chunks // RING - 1
        for b in range(RING):
            step(t_last, b, first_use=False, fire_ok=(b + LEAD < RING))

        for b in range(RING):
            wait_out(b)

    return sc_gather


def _mask_eos_kernel(ids_ref, mask_ref, eos_ref):
    ids = ids_ref[...]
    mask_ref[...] = ids == PAD_ID
    eos_ref[...] = (ids == EOS_ID).astype(jnp.float32)


def kernel(lookup_ids, embedding_matrix):
    b, s = lookup_ids.shape
    v, d = embedding_matrix.shape
    total = b * s

    ids_t = lookup_ids.T
    idx2 = ids_t.reshape(total // CHUNK, CHUNK)
    gather = _make_sc_gather(total, d)
    flat = gather(embedding_matrix, idx2)
    matrices = flat.reshape(s, b, d).transpose(1, 0, 2)

    mask_t, eos_t = pl.pallas_call(
        _mask_eos_kernel,
        out_shape=(
            jax.ShapeDtypeStruct((s, b), jnp.bool_),
            jax.ShapeDtypeStruct((s, b), jnp.float32),
        ),
    )(ids_t)

    return matrices, mask_t.T, eos_t.T, lookup_ids
